# Initial kernel scaffold; baseline (speedup 1.0000x reference)
import functools

import jax
import jax.numpy as jnp
from jax import lax
from jax.experimental import pallas as pl
from jax.experimental.pallas import tpu as pltpu
from jax.experimental.pallas import tpu_sc as plsc

N = 10000
E = 320000
NC = 2
NS = 16
NW = NC * NS
CH = 128
K = -(-E // (NW * CH))
EP = NW * K * CH
NN = 10048
R = NN // NS
EPS = 1e-5

_mesh = plsc.VectorSubcoreMesh(core_axis_name="c", subcore_axis_name="s")


def _make_sc_scatter(D):
  @functools.partial(
      pl.kernel,
      out_type=jax.ShapeDtypeStruct((NC, NN, D), jnp.float32),
      mesh=_mesh,
      scratch_types=[
          pltpu.VMEM((K * CH,), jnp.int32),
          pltpu.VMEM((K, CH), jnp.int32),
          pltpu.VMEM((CH, D), jnp.float32),
          pltpu.VMEM_SHARED((NN, D), jnp.float32),
          pltpu.SemaphoreType.DMA,
      ],
  )
  def sc_scatter(table, srcf, dst2, zeros, out_p, src_v, dst_v, rows, acc,
                 sem):
    c = lax.axis_index("c")
    s = lax.axis_index("s")
    w = s * NC + c
    pltpu.sync_copy(srcf.at[pl.ds(w * (K * CH), K * CH)], src_v)
    pltpu.sync_copy(dst2.at[pl.ds(w * K, K)], dst_v)
    pltpu.sync_copy(zeros.at[pl.ds(s * R, R)], acc.at[pl.ds(s * R, R)])
    plsc.subcore_barrier()

    def body(j, carry):
      pltpu.async_copy(table.at[src_v.at[pl.ds(j * CH, CH)]], rows,
                       sem).wait()
      pltpu.sync_copy(rows, acc.at[dst_v.at[j]], add=True)
      return carry

    lax.fori_loop(0, K, body, 0)
    plsc.subcore_barrier()
    pltpu.sync_copy(acc.at[pl.ds(s * R, R)], out_p.at[c, pl.ds(s * R, R)])

  return sc_scatter


_sc_scatter_128 = _make_sc_scatter(128)
_sc_scatter_64 = _make_sc_scatter(64)


@functools.partial(
    pl.kernel,
    out_type=jax.ShapeDtypeStruct((NC, NN, 16), jnp.float32),
    mesh=_mesh,
    scratch_types=[
        pltpu.VMEM((K, CH), jnp.int32),
        pltpu.VMEM((CH, 16), jnp.float32),
        pltpu.VMEM_SHARED((NN, 16), jnp.float32),
    ],
)
def _sc_deg(dst2, ones, zeros, out_p, dst_v, ones_v, acc):
  c = lax.axis_index("c")
  s = lax.axis_index("s")
  w = s * NC + c
  pltpu.sync_copy(dst2.at[pl.ds(w * K, K)], dst_v)
  pltpu.sync_copy(ones, ones_v)
  pltpu.sync_copy(zeros.at[pl.ds(s * R, R)], acc.at[pl.ds(s * R, R)])
  plsc.subcore_barrier()

  def body(j, carry):
    pltpu.sync_copy(ones_v, acc.at[dst_v.at[j]], add=True)
    return carry

  lax.fori_loop(0, K, body, 0)
  plsc.subcore_barrier()
  pltpu.sync_copy(acc.at[pl.ds(s * R, R)], out_p.at[c, pl.ds(s * R, R)])


def _tc1_body(x_ref, w_ref, degp_ref, g_ref, dinv_ref):
  deg = degp_ref[0, :N, 0:1] + degp_ref[1, :N, 0:1] + 1.0
  dinv = lax.rsqrt(deg)
  h = jnp.dot(x_ref[...], w_ref[...], preferred_element_type=jnp.float32)
  g_ref[:N, :] = h * dinv
  g_ref[N:, :] = jnp.zeros((NN - N, g_ref.shape[1]), jnp.float32)
  dinv_ref[...] = dinv


def _tc2_body(g_ref, s_ref, dinv_ref, b_ref, gam_ref, bet_ref, w_ref,
              out_ref):
  dinv = dinv_ref[...]
  z = dinv * (s_ref[0, :N, :] + s_ref[1, :N, :] + g_ref[:N, :]) + b_ref[...]
  mean = jnp.mean(z, axis=0, keepdims=True)
  var = jnp.mean((z - mean) ** 2, axis=0, keepdims=True)
  y = (z - mean) * lax.rsqrt(var + EPS) * gam_ref[...] + bet_ref[...]
  y = jax.nn.sigmoid(y)
  h2 = jnp.dot(y, w_ref[...], preferred_element_type=jnp.float32)
  out_ref[:N, :] = h2 * dinv
  out_ref[N:, :] = jnp.zeros((NN - N, out_ref.shape[1]), jnp.float32)


def _tc3_body(g_ref, s_ref, dinv_ref, b_ref, gam_ref, bet_ref, out_ref):
  dinv = dinv_ref[...]
  z = dinv * (s_ref[0, :N, :] + s_ref[1, :N, :] + g_ref[:N, :]) + b_ref[...]
  mean = jnp.mean(z, axis=0, keepdims=True)
  var = jnp.mean((z - mean) ** 2, axis=0, keepdims=True)
  out_ref[...] = (z - mean) * lax.rsqrt(var + EPS) * gam_ref[...] + bet_ref[
      ...]


def kernel(x, edge_index, W1, b1, gamma1, beta1, W2, b2, gamma2, beta2):
  ei = edge_index.astype(jnp.int32)
  pad = jnp.full((EP - E,), N, jnp.int32)
  srcp = jnp.concatenate([ei[0], pad])
  dst2 = jnp.concatenate([ei[1], pad]).reshape(NW * K, CH)

  z128 = jnp.zeros((NN, 128), jnp.float32)
  z64 = jnp.zeros((NN, 64), jnp.float32)
  z16 = jnp.zeros((NN, 16), jnp.float32)
  ones16 = jnp.ones((CH, 16), jnp.float32)

  deg_p = _sc_deg(dst2, ones16, z16)

  g1, dinv = pl.pallas_call(
      _tc1_body,
      out_shape=(jax.ShapeDtypeStruct((NN, 128), jnp.float32),
                 jax.ShapeDtypeStruct((N, 1), jnp.float32)),
  )(x, W1, deg_p)

  s1 = _sc_scatter_128(g1, srcp, dst2, z128)

  g2 = pl.pallas_call(
      _tc2_body,
      out_shape=jax.ShapeDtypeStruct((NN, 64), jnp.float32),
  )(g1, s1, dinv, b1.reshape(1, -1), gamma1.reshape(1, -1),
    beta1.reshape(1, -1), W2)

  s2 = _sc_scatter_64(g2, srcp, dst2, z64)

  out = pl.pallas_call(
      _tc3_body,
      out_shape=jax.ShapeDtypeStruct((N, 64), jnp.float32),
  )(g2, s2, dinv, b2.reshape(1, -1), gamma2.reshape(1, -1),
    beta2.reshape(1, -1))
  return out


# speedup vs baseline: 13.4434x; 13.4434x over previous
import functools

import jax
import jax.numpy as jnp
from jax import lax
from jax.experimental import pallas as pl
from jax.experimental.pallas import tpu as pltpu
from jax.experimental.pallas import tpu_sc as plsc

N = 10000
E = 320000
NC = 2
NS = 16
NW = NC * NS
CH = 128
K = -(-E // (NW * CH))
EP = NW * K * CH
NN = 10112
R = NN // NS
EPS = 1e-5

_mesh = plsc.VectorSubcoreMesh(core_axis_name="c", subcore_axis_name="s")
_sc_params = pltpu.CompilerParams(use_tc_tiling_on_sc=False)


def _make_sc_scatter(D):
  @functools.partial(
      pl.kernel,
      out_type=jax.ShapeDtypeStruct((NC, NN, D), jnp.float32),
      mesh=_mesh,
      compiler_params=_sc_params,
      scratch_types=[
          pltpu.VMEM((K * CH,), jnp.int32),
          pltpu.VMEM((CH,), jnp.int32),
          pltpu.VMEM((CH, D), jnp.float32),
          pltpu.VMEM_SHARED((NN, D), jnp.float32),
          pltpu.SemaphoreType.DMA,
      ],
  )
  def sc_scatter(table, srcf, dstf, zeros, out_p, src_v, dst_c, rows, acc,
                 sem):
    c = lax.axis_index("c")
    s = lax.axis_index("s")
    w = s * NC + c
    base = w * (K * CH)
    pltpu.sync_copy(srcf.at[pl.ds(base, K * CH)], src_v)
    pltpu.sync_copy(zeros.at[pl.ds(s * R, R)], acc.at[pl.ds(s * R, R)])
    plsc.subcore_barrier()

    def body(j, carry):
      pltpu.sync_copy(dstf.at[pl.ds(base + j * CH, CH)], dst_c)
      pltpu.async_copy(table.at[src_v.at[pl.ds(j * CH, CH)]], rows,
                       sem).wait()
      pltpu.sync_copy(rows, acc.at[dst_c], add=True)
      return carry

    lax.fori_loop(0, K, body, 0)
    plsc.subcore_barrier()
    pltpu.sync_copy(acc.at[pl.ds(s * R, R)], out_p.at[c, pl.ds(s * R, R)])

  return sc_scatter


_sc_scatter_128 = _make_sc_scatter(128)
_sc_scatter_64 = _make_sc_scatter(64)


@functools.partial(
    pl.kernel,
    out_type=jax.ShapeDtypeStruct((NC, NN, 16), jnp.float32),
    mesh=_mesh,
    compiler_params=_sc_params,
    scratch_types=[
        pltpu.VMEM((CH,), jnp.int32),
        pltpu.VMEM((CH, 16), jnp.float32),
        pltpu.VMEM_SHARED((NN, 16), jnp.float32),
    ],
)
def _sc_deg(dstf, ones, zeros, out_p, dst_c, ones_v, acc):
  c = lax.axis_index("c")
  s = lax.axis_index("s")
  w = s * NC + c
  base = w * (K * CH)
  pltpu.sync_copy(ones, ones_v)
  pltpu.sync_copy(zeros.at[pl.ds(s * R, R)], acc.at[pl.ds(s * R, R)])
  plsc.subcore_barrier()

  def body(j, carry):
    pltpu.sync_copy(dstf.at[pl.ds(base + j * CH, CH)], dst_c)
    pltpu.sync_copy(ones_v, acc.at[dst_c], add=True)
    return carry

  lax.fori_loop(0, K, body, 0)
  plsc.subcore_barrier()
  pltpu.sync_copy(acc.at[pl.ds(s * R, R)], out_p.at[c, pl.ds(s * R, R)])


def _tc1_body(x_ref, w_ref, degp_ref, g_ref, dinv_ref):
  deg = degp_ref[0, :N, 0:1] + degp_ref[1, :N, 0:1] + 1.0
  dinv = lax.rsqrt(deg)
  h = jnp.dot(x_ref[...], w_ref[...], preferred_element_type=jnp.float32)
  g_ref[:N, :] = h * dinv
  g_ref[N:, :] = jnp.zeros((NN - N, g_ref.shape[1]), jnp.float32)
  dinv_ref[...] = dinv


def _tc2_body(g_ref, s_ref, dinv_ref, b_ref, gam_ref, bet_ref, w_ref,
              out_ref):
  dinv = dinv_ref[...]
  z = dinv * (s_ref[0, :N, :] + s_ref[1, :N, :] + g_ref[:N, :]) + b_ref[...]
  mean = jnp.mean(z, axis=0, keepdims=True)
  var = jnp.mean((z - mean) ** 2, axis=0, keepdims=True)
  y = (z - mean) * lax.rsqrt(var + EPS) * gam_ref[...] + bet_ref[...]
  y = jax.nn.sigmoid(y)
  h2 = jnp.dot(y, w_ref[...], preferred_element_type=jnp.float32)
  out_ref[:N, :] = h2 * dinv
  out_ref[N:, :] = jnp.zeros((NN - N, out_ref.shape[1]), jnp.float32)


def _tc3_body(g_ref, s_ref, dinv_ref, b_ref, gam_ref, bet_ref, out_ref):
  dinv = dinv_ref[...]
  z = dinv * (s_ref[0, :N, :] + s_ref[1, :N, :] + g_ref[:N, :]) + b_ref[...]
  mean = jnp.mean(z, axis=0, keepdims=True)
  var = jnp.mean((z - mean) ** 2, axis=0, keepdims=True)
  out_ref[...] = (z - mean) * lax.rsqrt(var + EPS) * gam_ref[...] + bet_ref[
      ...]


def kernel(x, edge_index, W1, b1, gamma1, beta1, W2, b2, gamma2, beta2):
  ei = edge_index.astype(jnp.int32)
  pad = jnp.full((EP - E,), N, jnp.int32)
  srcp = jnp.concatenate([ei[0], pad])
  dstp = jnp.concatenate([ei[1], pad])

  z128 = jnp.zeros((NN, 128), jnp.float32)
  z64 = jnp.zeros((NN, 64), jnp.float32)
  z16 = jnp.zeros((NN, 16), jnp.float32)
  ones16 = jnp.ones((CH, 16), jnp.float32)

  deg_p = _sc_deg(dstp, ones16, z16)

  g1, dinv = pl.pallas_call(
      _tc1_body,
      out_shape=(jax.ShapeDtypeStruct((NN, 128), jnp.float32),
                 jax.ShapeDtypeStruct((N, 1), jnp.float32)),
  )(x, W1, deg_p)

  s1 = _sc_scatter_128(g1, srcp, dstp, z128)

  g2 = pl.pallas_call(
      _tc2_body,
      out_shape=jax.ShapeDtypeStruct((NN, 64), jnp.float32),
  )(g1, s1, dinv, b1.reshape(1, -1), gamma1.reshape(1, -1),
    beta1.reshape(1, -1), W2)

  s2 = _sc_scatter_64(g2, srcp, dstp, z64)

  out = pl.pallas_call(
      _tc3_body,
      out_shape=jax.ShapeDtypeStruct((N, 64), jnp.float32),
  )(g2, s2, dinv, b2.reshape(1, -1), gamma2.reshape(1, -1),
    beta2.reshape(1, -1))
  return out
